# baseline (speedup 1.0000x reference)
import functools

import jax
import jax.numpy as jnp
from jax import lax
from jax.experimental import pallas as pl
from jax.experimental.pallas import tpu as pltpu
from jax.experimental.pallas import tpu_sc as plsc

H = 128
N = 10000
E = 320000
NIDX_PAD = 5120
NC = 2
NS = 16
CHUNK = 128
GCHUNK = E // CHUNK
NPT = GCHUNK // NS
NSPARE = GCHUNK - NPT * NS

@functools.lru_cache(maxsize=None)
def _mesh():
    return plsc.VectorSubcoreMesh(core_axis_name="c", subcore_axis_name="s")


NBUF = 3
DEPTH = 2
NIBUF = 6
IDEPTH = 5


def _segsum_body(y, src_all, dst_all, zrows, out, src_vs, dst_vs, rows_vs,
                 acc, isems, dsems, gsems, ssems):
    c = lax.axis_index("c")
    s = lax.axis_index("s")
    rpt = 624
    for z in range(3):
        pltpu.sync_copy(zrows, acc.at[pl.ds(s * rpt + z * 208, 208)])

    @pl.when(s == 0)
    def _():
        pltpu.sync_copy(zrows.at[pl.ds(0, 16)], acc.at[pl.ds(NS * rpt, 16)])

    plsc.subcore_barrier()

    ebase = c * E + s * CHUNK

    def start_idx(k, bi):
        off = ebase + k * (NS * CHUNK)
        pltpu.async_copy(src_all.at[pl.ds(off, CHUNK)], src_vs[bi], isems[bi])
        pltpu.async_copy(dst_all.at[pl.ds(off, CHUNK)], dst_vs[bi], dsems[bi])

    def start_gather(bi, br):
        pltpu.make_async_copy(src_all.at[pl.ds(0, CHUNK)],
                              src_vs[bi], isems[bi]).wait()
        pltpu.async_copy(y.at[src_vs[bi]], rows_vs[br], gsems[br])

    def start_scatter(bi, br):
        pltpu.make_async_copy(y.at[src_vs[0]], rows_vs[br], gsems[br]).wait()
        pltpu.make_async_copy(dst_all.at[pl.ds(0, CHUNK)],
                              dst_vs[bi], dsems[bi]).wait()
        pltpu.async_copy(rows_vs[br], acc.at[dst_vs[bi]], ssems[br], add=True)

    def wait_scatter(br):
        pltpu.make_async_copy(rows_vs[br], acc.at[dst_vs[0]], ssems[br]).wait()

    for k0 in range(IDEPTH):
        start_idx(k0, k0)
    for k0 in range(DEPTH):
        start_gather(k0, k0)

    def outer(g, carry):
        for bb in range(NIBUF):
            k = g * NIBUF + bb

            @pl.when(k + IDEPTH < NPT)
            def _():
                start_idx(k + IDEPTH, (bb + IDEPTH) % NIBUF)

            @pl.when(k + DEPTH < NPT)
            def _():
                @pl.when(k >= 1)
                def _():
                    wait_scatter((bb + DEPTH) % NBUF)
                start_gather((bb + DEPTH) % NIBUF, (bb + DEPTH) % NBUF)

            start_scatter(bb, bb % NBUF)
        return carry

    lax.fori_loop(0, NPT // NIBUF, outer, 0)
    for b in range(NBUF):
        wait_scatter((NPT + b) % NBUF)

    @pl.when(s < NSPARE)
    def _():
        start_idx(NPT, 0)
        start_gather(0, 0)
        start_scatter(0, 0)
        wait_scatter(0)

    plsc.subcore_barrier()
    pltpu.sync_copy(
        acc.at[pl.ds(s * rpt, rpt)],
        out.at[c, pl.ds(s * rpt, rpt)],
    )

    @pl.when(s == 0)
    def _():
        pltpu.sync_copy(
            acc.at[pl.ds(NS * rpt, 16)],
            out.at[c, pl.ds(NS * rpt, 16)],
        )


@functools.lru_cache(maxsize=None)
def _segsum():
    return pl.kernel(
        _segsum_body,
        out_type=jax.ShapeDtypeStruct((NC, N, H), jnp.float32),
        mesh=_mesh(),
        scratch_types=[
            [pltpu.VMEM((CHUNK,), jnp.int32) for _ in range(NIBUF)],
            [pltpu.VMEM((CHUNK,), jnp.int32) for _ in range(NIBUF)],
            [pltpu.VMEM((CHUNK, H), jnp.float32) for _ in range(NBUF)],
            pltpu.VMEM_SHARED((N, H), jnp.float32),
            [pltpu.SemaphoreType.DMA for _ in range(NIBUF)],
            [pltpu.SemaphoreType.DMA for _ in range(NIBUF)],
            [pltpu.SemaphoreType.DMA for _ in range(NBUF)],
            [pltpu.SemaphoreType.DMA for _ in range(NBUF)],
        ],
    )


GCH = 80


def _gather_body(xall, idx, out, idx_v, rows_v, sem):
    c = lax.axis_index("c")
    s = lax.axis_index("s")
    wid = s * NC + c
    per_w = NIDX_PAD // (NC * NS)
    for j in range(per_w // GCH):
        base = wid * per_w + j * GCH
        pltpu.sync_copy(idx.at[pl.ds(base, GCH)], idx_v)
        pltpu.async_copy(xall.at[idx_v], rows_v, sem).wait()
        pltpu.sync_copy(rows_v, out.at[pl.ds(base, GCH)])


@functools.lru_cache(maxsize=None)
def _gather():
    return pl.kernel(
        _gather_body,
        out_type=jax.ShapeDtypeStruct((NIDX_PAD, 5 * H), jnp.float32),
        mesh=_mesh(),
        scratch_types=[
            pltpu.VMEM((GCH,), jnp.int32),
            pltpu.VMEM((GCH, 5 * H), jnp.float32),
            pltpu.SemaphoreType.DMA,
        ],
    )


def _enc_kernel(nf, w1, b1, w2, b2, out_nf, out_y):
    h = jnp.maximum(nf[...] @ w1[...] + b1[...], 0.0)
    h = h @ w2[...] + b2[...]
    out_nf[...] = h
    out_y[...] = jnp.maximum(h, 0.0)


def _mm16(a, w):
    return jnp.dot(a.astype(jnp.bfloat16), w.astype(jnp.bfloat16),
                   preferred_element_type=jnp.float32)


def _conv_post(x, t, W1, b1, W2, b2, gamma, beta, eps):
    h = (1.0 + eps) * x + t
    h = jnp.maximum(_mm16(h, W1) + b1, 0.0)
    h = jnp.maximum(_mm16(h, W2) + b2, 0.0)
    mean = jnp.mean(h, axis=0)
    var = jnp.mean((h - mean) ** 2, axis=0)
    return (h - mean) / jnp.sqrt(var + 1e-5) * gamma + beta


def _layer_kernel(x, tmp,
                  w11, b11, w12, b12, g1, be1, e1,
                  w21, b21, w22, b22, g2, be2, e2,
                  jw1, jb1, jw2, jb2,
                  out_nf, out_y):
    xv = x[...]
    h1 = _conv_post(xv, tmp[0], w11[...], b11[...], w12[...], b12[...],
                    g1[...], be1[...], e1[...])
    h2 = _conv_post(xv, tmp[1], w21[...], b21[...], w22[...], b22[...],
                    g2[...], be2[...], e2[...])
    cat = jnp.concatenate([h1, h2], axis=-1)
    nf = _mm16(jnp.maximum(_mm16(cat, jw1[...]) + jb1[...], 0.0),
               jw2[...]) + jb2[...]
    out_nf[...] = nf
    out_y[...] = jnp.maximum(nf, 0.0)


def _head_kernel(g, w1, b1, w2, b2, w3, b3, w4, b4, out):
    x = jnp.maximum(g[...] @ w1[...] + b1[...], 0.0)
    x = jnp.maximum(x @ w2[...] + b2[...], 0.0)
    x = jnp.maximum(x @ w3[...] + b3[...], 0.0)
    x = x @ w4[...] + b4[...]
    m = jnp.max(x, axis=-1, keepdims=True)
    lse = m + jnp.log(jnp.sum(jnp.exp(x - m), axis=-1, keepdims=True))
    out[...] = x - lse


def _tc_call(body, out_shapes):
    return pl.pallas_call(body, out_shape=out_shapes)


def kernel(node_features, edge_index_1, edge_index_2, indices, params):
    p = params
    f32 = jnp.float32
    src_all = jnp.concatenate(
        [edge_index_1[0], edge_index_2[0]]).astype(jnp.int32)
    dst_all = jnp.concatenate(
        [edge_index_1[1], edge_index_2[1]]).astype(jnp.int32)
    zrows = jnp.zeros((208, H), f32)
    idx_pad = jnp.zeros((NIDX_PAD,), jnp.int32).at[:indices.shape[0]].set(
        indices.astype(jnp.int32))

    two_nh = (jax.ShapeDtypeStruct((N, H), f32), jax.ShapeDtypeStruct((N, H), f32))

    nf0, y = _tc_call(_enc_kernel, two_nh)(
        node_features, p["enc_W1"], p["enc_b1"], p["enc_W2"], p["enc_b2"])

    nfs = [nf0]
    nf = nf0
    for li in range(1, 5):
        c1 = p[f"c1{li}"]
        c2 = p[f"c2{li}"]
        tmp = _segsum()(y, src_all, dst_all, zrows)
        nf, y = _tc_call(_layer_kernel, two_nh)(
            nf, tmp,
            c1["W1"], c1["b1"], c1["W2"], c1["b2"], c1["gamma"], c1["beta"], c1["eps"],
            c2["W1"], c2["b1"], c2["W2"], c2["b2"], c2["gamma"], c2["beta"], c2["eps"],
            p["joint_W1"], p["joint_b1"], p["joint_W2"], p["joint_b2"])
        nfs.append(nf)

    xall = jnp.concatenate(nfs, axis=-1)
    g = _gather()(xall, idx_pad)
    out = _tc_call(_head_kernel, jax.ShapeDtypeStruct((NIDX_PAD, 2), f32))(
        g, p["lin1_W"], p["lin1_b"], p["lin2_W"], p["lin2_b"],
        p["lin3_W"], p["lin3_b"], p["lin4_W"], p["lin4_b"])
    return out[:indices.shape[0]]


# speedup vs baseline: 1.0096x; 1.0096x over previous
import functools

import jax
import jax.numpy as jnp
from jax import lax
from jax.experimental import pallas as pl
from jax.experimental.pallas import tpu as pltpu
from jax.experimental.pallas import tpu_sc as plsc

H = 128
N = 10000
E = 320000
NIDX_PAD = 5120
NC = 2
NS = 16
CHUNK = 128
GCHUNK = E // CHUNK
NPT = GCHUNK // NS
NSPARE = GCHUNK - NPT * NS

@functools.lru_cache(maxsize=None)
def _mesh():
    return plsc.VectorSubcoreMesh(core_axis_name="c", subcore_axis_name="s")


NBUF = 3
DEPTH = 2
NIBUF = 6
IDEPTH = 5


def _segsum_body(y, src_all, dst_all, zrows, out, src_vs, dst_vs, rows_vs,
                 acc, isems, dsems, gsems, ssems):
    c = lax.axis_index("c")
    s = lax.axis_index("s")
    rpt = 624
    for z in range(3):
        pltpu.sync_copy(zrows, acc.at[pl.ds(s * rpt + z * 208, 208)])

    @pl.when(s == 0)
    def _():
        pltpu.sync_copy(zrows.at[pl.ds(0, 16)], acc.at[pl.ds(NS * rpt, 16)])

    plsc.subcore_barrier()

    ebase = c * E + s * CHUNK

    def start_idx(k, bi):
        off = ebase + k * (NS * CHUNK)
        pltpu.async_copy(src_all.at[pl.ds(off, CHUNK)], src_vs[bi], isems[bi])
        pltpu.async_copy(dst_all.at[pl.ds(off, CHUNK)], dst_vs[bi], dsems[bi])

    def start_gather(bi, br):
        pltpu.make_async_copy(src_all.at[pl.ds(0, CHUNK)],
                              src_vs[bi], isems[bi]).wait()
        pltpu.async_copy(y.at[src_vs[bi]], rows_vs[br], gsems[br])

    def start_scatter(bi, br):
        pltpu.make_async_copy(y.at[src_vs[0]], rows_vs[br], gsems[br]).wait()
        pltpu.make_async_copy(dst_all.at[pl.ds(0, CHUNK)],
                              dst_vs[bi], dsems[bi]).wait()
        pltpu.async_copy(rows_vs[br], acc.at[dst_vs[bi]], ssems[br], add=True)

    def wait_scatter(br):
        pltpu.make_async_copy(rows_vs[br], acc.at[dst_vs[0]], ssems[br]).wait()

    for k0 in range(IDEPTH):
        start_idx(k0, k0)
    for k0 in range(DEPTH):
        start_gather(k0, k0)

    def outer(g, carry):
        for bb in range(NIBUF):
            k = g * NIBUF + bb

            @pl.when(k + IDEPTH < NPT)
            def _():
                start_idx(k + IDEPTH, (bb + IDEPTH) % NIBUF)

            @pl.when(k + DEPTH < NPT)
            def _():
                @pl.when(k >= 1)
                def _():
                    wait_scatter((bb + DEPTH) % NBUF)
                start_gather((bb + DEPTH) % NIBUF, (bb + DEPTH) % NBUF)

            start_scatter(bb, bb % NBUF)
        return carry

    lax.fori_loop(0, NPT // NIBUF, outer, 0)
    for b in range(NBUF):
        wait_scatter((NPT + b) % NBUF)

    @pl.when(s < NSPARE)
    def _():
        start_idx(NPT, 0)
        start_gather(0, 0)
        start_scatter(0, 0)
        wait_scatter(0)

    plsc.subcore_barrier()
    pltpu.sync_copy(
        acc.at[pl.ds(s * rpt, rpt)],
        out.at[c, pl.ds(s * rpt, rpt)],
    )

    @pl.when(s == 0)
    def _():
        pltpu.sync_copy(
            acc.at[pl.ds(NS * rpt, 16)],
            out.at[c, pl.ds(NS * rpt, 16)],
        )


@functools.lru_cache(maxsize=None)
def _segsum():
    return pl.kernel(
        _segsum_body,
        out_type=jax.ShapeDtypeStruct((NC, N, H), jnp.float32),
        mesh=_mesh(),
        scratch_types=[
            [pltpu.VMEM((CHUNK,), jnp.int32) for _ in range(NIBUF)],
            [pltpu.VMEM((CHUNK,), jnp.int32) for _ in range(NIBUF)],
            [pltpu.VMEM((CHUNK, H), jnp.float32) for _ in range(NBUF)],
            pltpu.VMEM_SHARED((N, H), jnp.float32),
            [pltpu.SemaphoreType.DMA for _ in range(NIBUF)],
            [pltpu.SemaphoreType.DMA for _ in range(NIBUF)],
            [pltpu.SemaphoreType.DMA for _ in range(NBUF)],
            [pltpu.SemaphoreType.DMA for _ in range(NBUF)],
        ],
    )


GCH = 80


def _gather_body(xall, idx, out, idx_v, rows_v, sem):
    c = lax.axis_index("c")
    s = lax.axis_index("s")
    wid = s * NC + c
    per_w = NIDX_PAD // (NC * NS)
    for j in range(per_w // GCH):
        base = wid * per_w + j * GCH
        pltpu.sync_copy(idx.at[pl.ds(base, GCH)], idx_v)
        pltpu.async_copy(xall.at[idx_v], rows_v, sem).wait()
        pltpu.sync_copy(rows_v, out.at[pl.ds(base, GCH)])


@functools.lru_cache(maxsize=None)
def _gather():
    return pl.kernel(
        _gather_body,
        out_type=jax.ShapeDtypeStruct((NIDX_PAD, 5 * H), jnp.float32),
        mesh=_mesh(),
        scratch_types=[
            pltpu.VMEM((GCH,), jnp.int32),
            pltpu.VMEM((GCH, 5 * H), jnp.float32),
            pltpu.SemaphoreType.DMA,
        ],
    )


def _enc_kernel(nf, w1, b1, w2, b2, out_nf, out_y):
    h = jnp.maximum(nf[...] @ w1[...] + b1[...], 0.0)
    h = h @ w2[...] + b2[...]
    out_nf[...] = h
    out_y[...] = jnp.maximum(h, 0.0)


def _conv_post(x, t, W1, b1, W2, b2, gamma, beta, eps):
    h = (1.0 + eps) * x + t
    h = jnp.maximum(h @ W1 + b1, 0.0)
    h = jnp.maximum(h @ W2 + b2, 0.0)
    mean = jnp.mean(h, axis=0)
    var = jnp.mean((h - mean) ** 2, axis=0)
    return (h - mean) / jnp.sqrt(var + 1e-5) * gamma + beta


def _layer_kernel(x, tmp,
                  w11, b11, w12, b12, g1, be1, e1,
                  w21, b21, w22, b22, g2, be2, e2,
                  jw1, jb1, jw2, jb2,
                  out_nf, out_y):
    xv = x[...]
    h1 = _conv_post(xv, tmp[0], w11[...], b11[...], w12[...], b12[...],
                    g1[...], be1[...], e1[...])
    h2 = _conv_post(xv, tmp[1], w21[...], b21[...], w22[...], b22[...],
                    g2[...], be2[...], e2[...])
    cat = jnp.concatenate([h1, h2], axis=-1)
    nf = jnp.maximum(cat @ jw1[...] + jb1[...], 0.0) @ jw2[...] + jb2[...]
    out_nf[...] = nf
    out_y[...] = jnp.maximum(nf, 0.0)


def _head_kernel(g, w1, b1, w2, b2, w3, b3, w4, b4, out):
    x = jnp.maximum(g[...] @ w1[...] + b1[...], 0.0)
    x = jnp.maximum(x @ w2[...] + b2[...], 0.0)
    x = jnp.maximum(x @ w3[...] + b3[...], 0.0)
    x = x @ w4[...] + b4[...]
    m = jnp.max(x, axis=-1, keepdims=True)
    lse = m + jnp.log(jnp.sum(jnp.exp(x - m), axis=-1, keepdims=True))
    out[...] = x - lse


def _tc_call(body, out_shapes):
    return pl.pallas_call(body, out_shape=out_shapes)


def kernel(node_features, edge_index_1, edge_index_2, indices, params):
    p = params
    f32 = jnp.float32
    src_all = jnp.concatenate(
        [edge_index_1[0], edge_index_2[0]]).astype(jnp.int32)
    dst_all = jnp.concatenate(
        [edge_index_1[1], edge_index_2[1]]).astype(jnp.int32)
    zrows = jnp.zeros((208, H), f32)
    idx_pad = jnp.zeros((NIDX_PAD,), jnp.int32).at[:indices.shape[0]].set(
        indices.astype(jnp.int32))

    two_nh = (jax.ShapeDtypeStruct((N, H), f32), jax.ShapeDtypeStruct((N, H), f32))

    nf0, y = _tc_call(_enc_kernel, two_nh)(
        node_features, p["enc_W1"], p["enc_b1"], p["enc_W2"], p["enc_b2"])

    nfs = [nf0]
    nf = nf0
    for li in range(1, 5):
        c1 = p[f"c1{li}"]
        c2 = p[f"c2{li}"]
        tmp = _segsum()(y, src_all, dst_all, zrows)
        nf, y = _tc_call(_layer_kernel, two_nh)(
            nf, tmp,
            c1["W1"], c1["b1"], c1["W2"], c1["b2"], c1["gamma"], c1["beta"], c1["eps"],
            c2["W1"], c2["b1"], c2["W2"], c2["b2"], c2["gamma"], c2["beta"], c2["eps"],
            p["joint_W1"], p["joint_b1"], p["joint_W2"], p["joint_b2"])
        nfs.append(nf)

    xall = jnp.concatenate(nfs, axis=-1)
    g = _gather()(xall, idx_pad)
    out = _tc_call(_head_kernel, jax.ShapeDtypeStruct((NIDX_PAD, 2), f32))(
        g, p["lin1_W"], p["lin1_b"], p["lin2_W"], p["lin2_b"],
        p["lin3_W"], p["lin3_b"], p["lin4_W"], p["lin4_b"])
    return out[:indices.shape[0]]


# speedup vs baseline: 1.0444x; 1.0345x over previous
import functools

import jax
import jax.numpy as jnp
from jax import lax
from jax.experimental import pallas as pl
from jax.experimental.pallas import tpu as pltpu
from jax.experimental.pallas import tpu_sc as plsc

H = 128
N = 10000
E = 320000
NIDX_PAD = 5120
NC = 2
NS = 16
CHUNK = 128
GCHUNK = E // CHUNK
NPT = GCHUNK // NS
NSPARE = GCHUNK - NPT * NS

@functools.lru_cache(maxsize=None)
def _mesh():
    return plsc.VectorSubcoreMesh(core_axis_name="c", subcore_axis_name="s")


NBUF = 3
DEPTH = 2
NIBUF = 6
IDEPTH = 5


def _segsum_body(y, eidx, zrows, out, ij_vs, rows_vs,
                 acc, isems, gsems, ssems):
    c = lax.axis_index("c")
    s = lax.axis_index("s")
    rpt = 624

    def start_idx(k, bi):
        off = s * CHUNK + k * (NS * CHUNK)
        pltpu.async_copy(eidx.at[c, pl.ds(0, 2), pl.ds(off, CHUNK)],
                         ij_vs[bi], isems[bi])

    def start_gather(bi, br):
        pltpu.make_async_copy(eidx.at[0, pl.ds(0, 2), pl.ds(0, CHUNK)],
                              ij_vs[bi], isems[bi]).wait()
        pltpu.async_copy(y.at[ij_vs[bi].at[0]], rows_vs[br], gsems[br])

    def start_scatter(bi, br):
        pltpu.make_async_copy(y.at[ij_vs[0].at[0]], rows_vs[br],
                              gsems[br]).wait()
        pltpu.async_copy(rows_vs[br], acc.at[ij_vs[bi].at[1]], ssems[br],
                         add=True)

    def wait_scatter(br):
        pltpu.make_async_copy(rows_vs[br], acc.at[ij_vs[0].at[1]],
                              ssems[br]).wait()

    for k0 in range(IDEPTH):
        start_idx(k0, k0)
    for k0 in range(DEPTH):
        start_gather(k0, k0)
    for z in range(3):
        pltpu.sync_copy(zrows, acc.at[pl.ds(s * rpt + z * 208, 208)])

    @pl.when(s == 0)
    def _():
        pltpu.sync_copy(zrows.at[pl.ds(0, 16)], acc.at[pl.ds(NS * rpt, 16)])

    plsc.subcore_barrier()

    def outer(g, carry):
        for bb in range(NIBUF):
            k = g * NIBUF + bb

            @pl.when((k >= 1) & (k + DEPTH < NPT))
            def _():
                wait_scatter((bb + DEPTH) % NBUF)

            @pl.when(k + IDEPTH < NPT)
            def _():
                start_idx(k + IDEPTH, (bb + IDEPTH) % NIBUF)

            @pl.when(k + DEPTH < NPT)
            def _():
                start_gather((bb + DEPTH) % NIBUF, (bb + DEPTH) % NBUF)

            start_scatter(bb, bb % NBUF)
        return carry

    lax.fori_loop(0, NPT // NIBUF, outer, 0)
    for b in range(NBUF):
        wait_scatter((NPT + b) % NBUF)

    @pl.when(s < NSPARE)
    def _():
        start_idx(NPT, 0)
        start_gather(0, 0)
        start_scatter(0, 0)
        wait_scatter(0)

    plsc.subcore_barrier()
    pltpu.sync_copy(
        acc.at[pl.ds(s * rpt, rpt)],
        out.at[c, pl.ds(s * rpt, rpt)],
    )

    @pl.when(s == 0)
    def _():
        pltpu.sync_copy(
            acc.at[pl.ds(NS * rpt, 16)],
            out.at[c, pl.ds(NS * rpt, 16)],
        )


@functools.lru_cache(maxsize=None)
def _segsum():
    return pl.kernel(
        _segsum_body,
        out_type=jax.ShapeDtypeStruct((NC, N, H), jnp.float32),
        mesh=_mesh(),
        scratch_types=[
            [pltpu.VMEM((2, CHUNK), jnp.int32) for _ in range(NIBUF)],
            [pltpu.VMEM((CHUNK, H), jnp.float32) for _ in range(NBUF)],
            pltpu.VMEM_SHARED((N, H), jnp.float32),
            [pltpu.SemaphoreType.DMA for _ in range(NIBUF)],
            [pltpu.SemaphoreType.DMA for _ in range(NBUF)],
            [pltpu.SemaphoreType.DMA for _ in range(NBUF)],
        ],
    )


GCH = 80


def _gather_body(xall, idx, out, idx_vs, rows_vs, isms, gsms, osms):
    c = lax.axis_index("c")
    s = lax.axis_index("s")
    wid = s * NC + c
    per_w = NIDX_PAD // (NC * NS)
    nj = per_w // GCH
    for j in range(nj):
        pltpu.async_copy(idx.at[pl.ds(wid * per_w + j * GCH, GCH)],
                         idx_vs[j], isms[j])
    for j in range(nj):
        pltpu.make_async_copy(idx.at[pl.ds(0, GCH)], idx_vs[j], isms[j]).wait()
        pltpu.async_copy(xall.at[idx_vs[j]], rows_vs[j], gsms[j])
    for j in range(nj):
        base = wid * per_w + j * GCH
        pltpu.make_async_copy(xall.at[idx_vs[j]], rows_vs[j], gsms[j]).wait()
        pltpu.async_copy(rows_vs[j], out.at[pl.ds(base, GCH)], osms[j])
    for j in range(nj):
        pltpu.make_async_copy(rows_vs[j], out.at[pl.ds(0, GCH)],
                              osms[j]).wait()


@functools.lru_cache(maxsize=None)
def _gather():
    return pl.kernel(
        _gather_body,
        out_type=jax.ShapeDtypeStruct((NIDX_PAD, 5 * H), jnp.float32),
        mesh=_mesh(),
        scratch_types=[
            [pltpu.VMEM((GCH,), jnp.int32) for _ in range(2)],
            [pltpu.VMEM((GCH, 5 * H), jnp.float32) for _ in range(2)],
            [pltpu.SemaphoreType.DMA for _ in range(2)],
            [pltpu.SemaphoreType.DMA for _ in range(2)],
            [pltpu.SemaphoreType.DMA for _ in range(2)],
        ],
    )


def _enc_kernel(nf, w1, b1, w2, b2, out_nf, out_y):
    h = jnp.maximum(nf[...] @ w1[...] + b1[...], 0.0)
    h = h @ w2[...] + b2[...]
    out_nf[...] = h
    out_y[...] = jnp.maximum(h, 0.0)


def _conv_post(x, t, W1, b1, W2, b2, gamma, beta, eps):
    h = (1.0 + eps) * x + t
    h = jnp.maximum(h @ W1 + b1, 0.0)
    h = jnp.maximum(h @ W2 + b2, 0.0)
    mean = jnp.mean(h, axis=0)
    var = jnp.mean((h - mean) ** 2, axis=0)
    return (h - mean) / jnp.sqrt(var + 1e-5) * gamma + beta


def _layer_kernel(x, tmp,
                  w11, b11, w12, b12, g1, be1, e1,
                  w21, b21, w22, b22, g2, be2, e2,
                  jw1, jb1, jw2, jb2,
                  out_nf, out_y):
    xv = x[...]
    h1 = _conv_post(xv, tmp[0], w11[...], b11[...], w12[...], b12[...],
                    g1[...], be1[...], e1[...])
    h2 = _conv_post(xv, tmp[1], w21[...], b21[...], w22[...], b22[...],
                    g2[...], be2[...], e2[...])
    cat = jnp.concatenate([h1, h2], axis=-1)
    nf = jnp.maximum(cat @ jw1[...] + jb1[...], 0.0) @ jw2[...] + jb2[...]
    out_nf[...] = nf
    out_y[...] = jnp.maximum(nf, 0.0)


def _head_kernel(g, w1, b1, w2, b2, w3, b3, w4, b4, out):
    x = jnp.maximum(g[...] @ w1[...] + b1[...], 0.0)
    x = jnp.maximum(x @ w2[...] + b2[...], 0.0)
    x = jnp.maximum(x @ w3[...] + b3[...], 0.0)
    x = x @ w4[...] + b4[...]
    m = jnp.max(x, axis=-1, keepdims=True)
    lse = m + jnp.log(jnp.sum(jnp.exp(x - m), axis=-1, keepdims=True))
    out[...] = x - lse


def _tc_call(body, out_shapes):
    return pl.pallas_call(body, out_shape=out_shapes)


def kernel(node_features, edge_index_1, edge_index_2, indices, params):
    p = params
    f32 = jnp.float32
    eidx = jnp.stack([edge_index_1, edge_index_2]).astype(jnp.int32)
    zrows = jnp.zeros((208, H), f32)
    idx_pad = jnp.zeros((NIDX_PAD,), jnp.int32).at[:indices.shape[0]].set(
        indices.astype(jnp.int32))

    two_nh = (jax.ShapeDtypeStruct((N, H), f32), jax.ShapeDtypeStruct((N, H), f32))

    nf0, y = _tc_call(_enc_kernel, two_nh)(
        node_features, p["enc_W1"], p["enc_b1"], p["enc_W2"], p["enc_b2"])

    nfs = [nf0]
    nf = nf0
    for li in range(1, 5):
        c1 = p[f"c1{li}"]
        c2 = p[f"c2{li}"]
        tmp = _segsum()(y, eidx, zrows)
        nf, y = _tc_call(_layer_kernel, two_nh)(
            nf, tmp,
            c1["W1"], c1["b1"], c1["W2"], c1["b2"], c1["gamma"], c1["beta"], c1["eps"],
            c2["W1"], c2["b1"], c2["W2"], c2["b2"], c2["gamma"], c2["beta"], c2["eps"],
            p["joint_W1"], p["joint_b1"], p["joint_W2"], p["joint_b2"])
        nfs.append(nf)

    xall = jnp.concatenate(nfs, axis=-1)
    g = _gather()(xall, idx_pad)
    out = _tc_call(_head_kernel, jax.ShapeDtypeStruct((NIDX_PAD, 2), f32))(
        g, p["lin1_W"], p["lin1_b"], p["lin2_W"], p["lin2_b"],
        p["lin3_W"], p["lin3_b"], p["lin4_W"], p["lin4_b"])
    return out[:indices.shape[0]]
